# baseline (speedup 1.0000x reference)
import functools

import jax
import jax.numpy as jnp
from jax import lax
from jax.experimental import pallas as pl
from jax.experimental.pallas import tpu as pltpu
from jax.experimental.pallas import tpu_sc as plsc

NUM_EMBEDDINGS = 100000
DIM = 128
BATCH = 4096
HIST = 50

_info = plsc.get_sparse_core_info()
NC, NS = _info.num_cores, _info.num_subcores
NW = NC * NS
ROWS_PER_W = BATCH // NW
NPAIR = HIST // 2


@functools.partial(
    pl.kernel,
    mesh=plsc.VectorSubcoreMesh(core_axis_name="c", subcore_axis_name="s"),
    out_type=jax.ShapeDtypeStruct((HIST, BATCH, DIM), jnp.float32),
    scratch_types=[
        pltpu.VMEM((HIST, ROWS_PER_W), jnp.int32),
        pltpu.VMEM((2, ROWS_PER_W, DIM), jnp.float32),
        pltpu.VMEM((2, ROWS_PER_W, DIM), jnp.float32),
        pltpu.VMEM((2, ROWS_PER_W, DIM), jnp.float32),
        pltpu.SemaphoreType.DMA,
        pltpu.SemaphoreType.DMA,
        pltpu.SemaphoreType.DMA,
        pltpu.SemaphoreType.DMA,
        pltpu.SemaphoreType.DMA,
        pltpu.SemaphoreType.DMA,
        pltpu.SemaphoreType.DMA,
    ],
)
def _sc_gather(
    tab_hbm, idx_hbm, out_hbm, idx_v,
    r0, r1, r2, g0, g1, g2, w0, w1, w2, isem,
):
    wid = lax.axis_index("s") * NC + lax.axis_index("c")
    base = wid * ROWS_PER_W
    rows = (r0, r1, r2)
    gsem = (g0, g1, g2)
    wsem = (w0, w1, w2)
    pltpu.sync_copy(idx_hbm.at[pl.ds(0, 8), pl.ds(base, ROWS_PER_W)],
                    idx_v.at[pl.ds(0, 8)])
    pltpu.async_copy(idx_hbm.at[pl.ds(8, HIST - 8), pl.ds(base, ROWS_PER_W)],
                     idx_v.at[pl.ds(8, HIST - 8)], isem)

    def start_g(p, b):
        for j in (0, 1):
            pltpu.async_copy(
                tab_hbm.at[idx_v.at[2 * p + j]], rows[b].at[j], gsem[b]
            )

    def wait_g(p, b):
        for j in (0, 1):
            pltpu.make_async_copy(
                tab_hbm.at[idx_v.at[2 * p + j]], rows[b].at[j], gsem[b]
            ).wait()

    def start_w1(p, j, b):
        pltpu.async_copy(
            rows[b].at[j], out_hbm.at[2 * p + j, pl.ds(base, ROWS_PER_W)], wsem[b]
        )

    def wait_w(p, b):
        for j in (0, 1):
            pltpu.make_async_copy(
                rows[b].at[j], out_hbm.at[2 * p + j, pl.ds(base, ROWS_PER_W)],
                wsem[b],
            ).wait()

    start_g(0, 0)
    pltpu.make_async_copy(
        idx_hbm.at[pl.ds(8, HIST - 8), pl.ds(base, ROWS_PER_W)],
        idx_v.at[pl.ds(8, HIST - 8)], isem,
    ).wait()

    def body(o, carry):
        for k in range(3):
            p = 3 * o + k
            b = k
            nb = (k + 1) % 3

            @pl.when(p >= 2)
            def _():
                wait_w(p - 2, nb)

            start_g(p + 1, nb)
            for j in (0, 1):
                pltpu.make_async_copy(
                    tab_hbm.at[idx_v.at[2 * p + j]], rows[b].at[j], gsem[b]
                ).wait()
                start_w1(p, j, b)
        return carry

    lax.fori_loop(0, (NPAIR - 1) // 3, body, 0)
    p = NPAIR - 1
    for j in (0, 1):
        pltpu.make_async_copy(
            tab_hbm.at[idx_v.at[2 * p + j]], rows[0].at[j], gsem[0]
        ).wait()
        start_w1(p, j, 0)
    for q in (NPAIR - 3, NPAIR - 2, NPAIR - 1):
        wait_w(q, q % 3)


def kernel(x, table):
    xt = jnp.swapaxes(x.astype(jnp.int32), 0, 1)
    out = _sc_gather(table, xt)
    return jnp.swapaxes(out, 0, 1)
